# baseline (speedup 1.0000x reference)
import jax
import jax.numpy as jnp
from jax.experimental import pallas as pl
from jax.experimental.pallas import tpu as pltpu

_N = 131072
_D = 256
_BLK = 4096
_CHUNK = 512
_NB = _N // _BLK
_NCH = _BLK // _CHUNK


def _pipe_kernel(q_ref, k_hbm, v_hbm, o_hbm, kb, vb, ob, sk, sv, so):
    C = _CHUNK
    ii = jax.lax.broadcasted_iota(jnp.int32, (C, C), 0)
    jj = jax.lax.broadcasted_iota(jnp.int32, (C, C), 1)
    tri = jj <= ii
    q_row = q_ref[...]

    def start_in(b, slot):
        pltpu.make_async_copy(
            k_hbm.at[pl.ds(b * _BLK, _BLK), :], kb.at[slot], sk.at[slot]).start()
        pltpu.make_async_copy(
            v_hbm.at[pl.ds(b * _BLK, _BLK), :], vb.at[slot], sv.at[slot]).start()

    start_in(0, 0)
    start_in(1, 1)
    start_in(2, 2)

    def body(b, carry):
        cm, cu, cw = carry
        slot = jax.lax.rem(b, 4)
        oslot = jax.lax.rem(b, 2)

        @pl.when(b + 3 < _NB)
        def _():
            start_in(b + 3, jax.lax.rem(b + 3, 4))

        pltpu.make_async_copy(
            k_hbm.at[pl.ds(b * _BLK, _BLK), :], kb.at[slot], sk.at[slot]).wait()
        pltpu.make_async_copy(
            v_hbm.at[pl.ds(b * _BLK, _BLK), :], vb.at[slot], sv.at[slot]).wait()

        @pl.when(b >= 2)
        def _():
            pltpu.make_async_copy(
                ob.at[oslot], o_hbm.at[pl.ds((b - 2) * _BLK, _BLK), :],
                so.at[oslot]).wait()

        ks = kb.at[slot]
        vs = vb.at[slot]
        os_ = ob.at[oslot]
        s_row = jax.lax.dot_general(
            q_row, ks[...], (((1,), (1,)), ((), ())),
            preferred_element_type=jnp.float32)

        for c in range(_NCH):
            v_blk = vs[c * C:(c + 1) * C, :]
            S = jnp.broadcast_to(s_row[:, c * C:(c + 1) * C], (C, C))
            m_loc = jnp.max(jnp.where(tri, S, -jnp.inf), axis=1, keepdims=True)
            m_col = jnp.maximum(m_loc, cm)
            E = jnp.where(tri, jnp.exp(S - m_col), 0.0)
            ce = jnp.exp(cm - m_col)
            den = jnp.sum(E, axis=1, keepdims=True) + ce * cu
            num = jax.lax.dot_general(
                E, v_blk, (((1,), (0,)), ((), ())),
                preferred_element_type=jnp.float32)
            num = num + ce * cw
            os_[c * C:(c + 1) * C, :] = num / den
            cm = m_col[C - 1:C, :]
            cu = den[C - 1:C, :]
            cw = num[C - 1:C, :]

        pltpu.make_async_copy(
            os_, o_hbm.at[pl.ds(b * _BLK, _BLK), :], so.at[oslot]).start()
        return (cm, cu, cw)

    init = (jnp.full((1, 1), -jnp.inf, jnp.float32),
            jnp.zeros((1, 1), jnp.float32),
            jnp.zeros((1, _D), jnp.float32))
    jax.lax.fori_loop(0, _NB, body, init)

    for t in (_NB - 2, _NB - 1):
        pltpu.make_async_copy(
            ob.at[t % 2], o_hbm.at[pl.ds(t * _BLK, _BLK), :],
            so.at[t % 2]).wait()


def kernel(K, V, q):
    q2 = q.reshape(1, _D)
    return pl.pallas_call(
        _pipe_kernel,
        out_shape=jax.ShapeDtypeStruct((_N, _D), jnp.float32),
        in_specs=[
            pl.BlockSpec(memory_space=pltpu.VMEM),
            pl.BlockSpec(memory_space=pl.ANY),
            pl.BlockSpec(memory_space=pl.ANY),
        ],
        out_specs=pl.BlockSpec(memory_space=pl.ANY),
        scratch_shapes=[
            pltpu.VMEM((4, _BLK, _D), jnp.float32),
            pltpu.VMEM((4, _BLK, _D), jnp.float32),
            pltpu.VMEM((2, _BLK, _D), jnp.float32),
            pltpu.SemaphoreType.DMA((4,)),
            pltpu.SemaphoreType.DMA((4,)),
            pltpu.SemaphoreType.DMA((2,)),
        ],
        compiler_params=pltpu.CompilerParams(
            vmem_limit_bytes=58 * 1024 * 1024,
        ),
        name="aaren_scan_pipe",
    )(q2, K, V)


# speedup vs baseline: 1.0012x; 1.0012x over previous
import jax
import jax.numpy as jnp
from jax.experimental import pallas as pl
from jax.experimental.pallas import tpu as pltpu

_N = 131072
_D = 256
_BLK = 4096
_CHUNK = 512
_NB = _N // _BLK
_NCH = _BLK // _CHUNK


def _pipe_kernel(q_ref, k_hbm, v_hbm, o_hbm, kb, vb, ob, sk, sv, so):
    C = _CHUNK
    ii = jax.lax.broadcasted_iota(jnp.int32, (C, C), 0)
    jj = jax.lax.broadcasted_iota(jnp.int32, (C, C), 1)
    tri = jj <= ii
    q_row = q_ref[...]

    def start_in(b, slot):
        pltpu.make_async_copy(
            k_hbm.at[pl.ds(b * _BLK, _BLK), :], kb.at[slot], sk.at[slot]).start()
        pltpu.make_async_copy(
            v_hbm.at[pl.ds(b * _BLK, _BLK), :], vb.at[slot], sv.at[slot]).start()

    start_in(0, 0)
    start_in(1, 1)

    def body(b, carry):
        cm, cu, cw = carry
        slot = jax.lax.rem(b, 3)
        oslot = jax.lax.rem(b, 3)

        @pl.when(b + 2 < _NB)
        def _():
            start_in(b + 2, jax.lax.rem(b + 2, 3))

        pltpu.make_async_copy(
            k_hbm.at[pl.ds(b * _BLK, _BLK), :], kb.at[slot], sk.at[slot]).wait()
        pltpu.make_async_copy(
            v_hbm.at[pl.ds(b * _BLK, _BLK), :], vb.at[slot], sv.at[slot]).wait()

        @pl.when(b >= 3)
        def _():
            pltpu.make_async_copy(
                ob.at[oslot], o_hbm.at[pl.ds((b - 3) * _BLK, _BLK), :],
                so.at[oslot]).wait()

        ks = kb.at[slot]
        vs = vb.at[slot]
        os_ = ob.at[oslot]
        s_row = jax.lax.dot_general(
            q_row, ks[...], (((1,), (1,)), ((), ())),
            preferred_element_type=jnp.float32)

        for c in range(_NCH):
            v_blk = vs[c * C:(c + 1) * C, :]
            S = jnp.broadcast_to(s_row[:, c * C:(c + 1) * C], (C, C))
            m_loc = jnp.max(jnp.where(tri, S, -jnp.inf), axis=1, keepdims=True)
            m_col = jnp.maximum(m_loc, cm)
            E = jnp.where(tri, jnp.exp(S - m_col), 0.0)
            ce = jnp.exp(cm - m_col)
            den = jnp.sum(E, axis=1, keepdims=True) + ce * cu
            num = jax.lax.dot_general(
                E, v_blk, (((1,), (0,)), ((), ())),
                preferred_element_type=jnp.float32)
            num = num + ce * cw
            os_[c * C:(c + 1) * C, :] = num / den
            cm = m_col[C - 1:C, :]
            cu = den[C - 1:C, :]
            cw = num[C - 1:C, :]

        pltpu.make_async_copy(
            os_, o_hbm.at[pl.ds(b * _BLK, _BLK), :], so.at[oslot]).start()
        return (cm, cu, cw)

    init = (jnp.full((1, 1), -jnp.inf, jnp.float32),
            jnp.zeros((1, 1), jnp.float32),
            jnp.zeros((1, _D), jnp.float32))
    jax.lax.fori_loop(0, _NB, body, init)

    for t in (_NB - 3, _NB - 2, _NB - 1):
        pltpu.make_async_copy(
            ob.at[t % 3], o_hbm.at[pl.ds(t * _BLK, _BLK), :],
            so.at[t % 3]).wait()


def kernel(K, V, q):
    q2 = q.reshape(1, _D)
    return pl.pallas_call(
        _pipe_kernel,
        out_shape=jax.ShapeDtypeStruct((_N, _D), jnp.float32),
        in_specs=[
            pl.BlockSpec(memory_space=pltpu.VMEM),
            pl.BlockSpec(memory_space=pl.ANY),
            pl.BlockSpec(memory_space=pl.ANY),
        ],
        out_specs=pl.BlockSpec(memory_space=pl.ANY),
        scratch_shapes=[
            pltpu.VMEM((3, _BLK, _D), jnp.float32),
            pltpu.VMEM((3, _BLK, _D), jnp.float32),
            pltpu.VMEM((3, _BLK, _D), jnp.float32),
            pltpu.SemaphoreType.DMA((3,)),
            pltpu.SemaphoreType.DMA((3,)),
            pltpu.SemaphoreType.DMA((3,)),
        ],
        compiler_params=pltpu.CompilerParams(
            vmem_limit_bytes=58 * 1024 * 1024,
        ),
        name="aaren_scan_pipe",
    )(q2, K, V)
